# baseline (speedup 1.0000x reference)
import functools

import jax
import jax.numpy as jnp
import numpy as np
from jax.experimental import pallas as pl
from jax.experimental.pallas import tpu as pltpu

_SQRT3 = np.float32(np.sqrt(3.0))
_EPS = np.float32(1e-12)


def _node_kernel(fg_ref, w_ref, b_ref, src_ref, dst_ref, out_ref, acc_ref):
    T = fg_ref.shape[0]
    F = fg_ref.shape[1]
    step = pl.program_id(0)
    nsteps = pl.num_programs(0)
    partial = jnp.zeros(acc_ref.shape, jnp.float32)
    for j in range(F):
        partial += jax.lax.dot_general(
            w_ref[:, j, :].astype(jnp.bfloat16),
            fg_ref[:, j, :].astype(jnp.bfloat16),
            (((1,), (1,)), ((), ())),
            preferred_element_type=jnp.float32)

    @pl.when(step == 0)
    def _init():
        acc_ref[...] = partial

    @pl.when(step != 0)
    def _accum():
        acc_ref[...] += partial

    @pl.when(step == nsteps - 1)
    def _finalize():
        node = acc_ref[...] + b_ref[:, 0:1]
        node = node / jnp.maximum(
            jnp.sqrt(jnp.sum(node * node, axis=0, keepdims=True)), _EPS)
        EP = src_ref.shape[1]
        iota_e = jax.lax.broadcasted_iota(
            jnp.int32, (T, EP), 0).astype(jnp.float32)
        src_oh = (src_ref[0:1, :] == iota_e).astype(jnp.float32)
        dst_oh = (dst_ref[0:1, :] == iota_e).astype(jnp.float32)
        adj = jax.lax.dot_general(
            dst_oh, src_oh, (((1,), (1,)), ((), ())),
            preferred_element_type=jnp.float32,
            precision=jax.lax.Precision.HIGHEST)
        agg = jax.lax.dot_general(
            node, adj, (((1,), (1,)), ((), ())),
            preferred_element_type=jnp.float32,
            precision=jax.lax.Precision.HIGHEST)
        node = node + agg
        node = node / jnp.maximum(
            jnp.sqrt(jnp.sum(node * node, axis=0, keepdims=True)), _EPS)
        out_ref[...] = node


def _score_topk_kernel(inner_ref, outer_ref, node_ref, vals_ref, idx_ref):
    shell = outer_ref[...] * _SQRT3 - inner_ref[...]
    shell = shell / jnp.maximum(
        jnp.sqrt(jnp.sum(shell * shell, axis=1, keepdims=True)), _EPS)
    scores = jax.lax.dot_general(
        shell.astype(jnp.bfloat16), node_ref[...].astype(jnp.bfloat16),
        (((1,), (0,)), ((), ())),
        preferred_element_type=jnp.float32)
    bb = scores.shape[0]
    iota = jax.lax.broadcasted_iota(jnp.int32, scores.shape, 1)
    vals_cols, idx_cols = [], []
    work = scores
    for _ in range(5):
        m = jnp.max(work, axis=1, keepdims=True)
        hit = work == m
        sel = jnp.min(jnp.where(hit, iota, jnp.int32(1 << 20)),
                      axis=1, keepdims=True)
        vals_cols.append(m)
        idx_cols.append(sel)
        work = jnp.where(iota == sel, jnp.float32(-jnp.inf), work)
    vals_ref[...] = jnp.concatenate(
        vals_cols + [jnp.zeros((bb, 3), jnp.float32)], axis=1)
    idx_ref[...] = jnp.concatenate(
        idx_cols + [jnp.zeros((bb, 3), jnp.int32)], axis=1)


def kernel(inner_latent, outer_latent, face_grids, W, b, edge_index):
    B, D = inner_latent.shape
    T, F = face_grids.shape[0], face_grids.shape[1]

    E = edge_index.shape[1]
    EP = 512
    pad = jnp.full((EP - E,), -1, edge_index.dtype)
    src_f = jnp.broadcast_to(
        jnp.concatenate([edge_index[0], pad]).astype(jnp.float32)[None, :],
        (8, EP))
    dst_f = jnp.broadcast_to(
        jnp.concatenate([edge_index[1], pad]).astype(jnp.float32)[None, :],
        (8, EP))
    b2 = jnp.broadcast_to(b.reshape(D, 1), (D, 128))

    DB = 128
    W3 = W.reshape(D, F, D)
    node_t = pl.pallas_call(
        _node_kernel,
        grid=(D // DB,),
        in_specs=[
            pl.BlockSpec((T, F, DB), lambda i: (0, 0, i)),
            pl.BlockSpec((D, F, DB), lambda i: (0, 0, i)),
            pl.BlockSpec((D, 128), lambda i: (0, 0)),
            pl.BlockSpec((8, EP), lambda i: (0, 0)),
            pl.BlockSpec((8, EP), lambda i: (0, 0)),
        ],
        out_specs=pl.BlockSpec((D, T), lambda i: (0, 0)),
        out_shape=jax.ShapeDtypeStruct((D, T), jnp.float32),
        scratch_shapes=[pltpu.VMEM((D, T), jnp.float32)],
        compiler_params=pltpu.CompilerParams(
            dimension_semantics=("arbitrary",)),
    )(face_grids, W3, b2, src_f, dst_f)

    BB = 512
    nb = B // BB
    vals8, idx8 = pl.pallas_call(
        _score_topk_kernel,
        grid=(nb,),
        in_specs=[
            pl.BlockSpec((BB, D), lambda i: (i, 0)),
            pl.BlockSpec((BB, D), lambda i: (i, 0)),
            pl.BlockSpec((D, T), lambda i: (0, 0)),
        ],
        out_specs=[
            pl.BlockSpec((BB, 8), lambda i: (i, 0)),
            pl.BlockSpec((BB, 8), lambda i: (i, 0)),
        ],
        out_shape=[
            jax.ShapeDtypeStruct((B, 8), jnp.float32),
            jax.ShapeDtypeStruct((B, 8), jnp.int32),
        ],
        compiler_params=pltpu.CompilerParams(
            dimension_semantics=("parallel",)),
    )(inner_latent, outer_latent, node_t)

    return vals8[:, :5], idx8[:, :5]


# speedup vs baseline: 2.1295x; 2.1295x over previous
import functools

import jax
import jax.numpy as jnp
import numpy as np
from jax.experimental import pallas as pl
from jax.experimental.pallas import tpu as pltpu

_SQRT3 = np.float32(np.sqrt(3.0))
_EPS = np.float32(1e-12)


def _node_kernel(fg_ref, w_ref, b_ref, src_ref, dst_ref, out_ref, acc_ref):
    T = fg_ref.shape[0]
    F = fg_ref.shape[1]
    D = fg_ref.shape[2]
    step = pl.program_id(0)
    nsteps = pl.num_programs(0)
    partial = jnp.zeros(acc_ref.shape, jnp.float32)
    for j in range(F):
        partial += jax.lax.dot_general(
            w_ref[:, j * D:(j + 1) * D], fg_ref[:, j, :],
            (((1,), (1,)), ((), ())),
            preferred_element_type=jnp.float32)

    @pl.when(step == 0)
    def _init():
        acc_ref[...] = partial

    @pl.when(step != 0)
    def _accum():
        acc_ref[...] += partial

    @pl.when(step == nsteps - 1)
    def _finalize():
        node = acc_ref[...] + b_ref[:, 0:1]
        node = node / jnp.maximum(
            jnp.sqrt(jnp.sum(node * node, axis=0, keepdims=True)), _EPS)
        EP = src_ref.shape[1]
        iota_e = jax.lax.broadcasted_iota(
            jnp.int32, (T, EP), 0).astype(jnp.float32)
        src_oh = (src_ref[0:1, :] == iota_e).astype(jnp.float32)
        dst_oh = (dst_ref[0:1, :] == iota_e).astype(jnp.float32)
        adj = jax.lax.dot_general(
            dst_oh, src_oh, (((1,), (1,)), ((), ())),
            preferred_element_type=jnp.float32,
            precision=jax.lax.Precision.HIGHEST)
        agg = jax.lax.dot_general(
            node, adj, (((1,), (1,)), ((), ())),
            preferred_element_type=jnp.float32,
            precision=jax.lax.Precision.HIGHEST)
        node = node + agg
        node = node / jnp.maximum(
            jnp.sqrt(jnp.sum(node * node, axis=0, keepdims=True)), _EPS)
        out_ref[...] = node


def _score_topk_kernel(inner_ref, outer_ref, node_ref, ones_ref, vals_ref,
                       idx_ref):
    shell = outer_ref[...] * _SQRT3 - inner_ref[...]
    ssq = jax.lax.dot_general(
        shell * shell, ones_ref[0:1, :], (((1,), (1,)), ((), ())),
        preferred_element_type=jnp.float32,
        precision=jax.lax.Precision.HIGHEST)
    shell = shell / jnp.maximum(jnp.sqrt(ssq), _EPS)
    scores = jax.lax.dot_general(
        node_ref[...], shell, (((0,), (1,)), ((), ())),
        preferred_element_type=jnp.float32)
    t, bb = scores.shape
    iota = jax.lax.broadcasted_iota(jnp.int32, scores.shape, 0)
    vals_rows, idx_rows = [], []
    work = scores
    for _ in range(5):
        m = jnp.max(work, axis=0, keepdims=True)
        hit = work == m
        sel = jnp.min(jnp.where(hit, iota, jnp.int32(1 << 20)),
                      axis=0, keepdims=True)
        vals_rows.append(m)
        idx_rows.append(sel)
        work = jnp.where(iota == sel, jnp.float32(-jnp.inf), work)
    vals_ref[...] = jnp.concatenate(
        vals_rows + [jnp.zeros((3, bb), jnp.float32)], axis=0)
    idx_ref[...] = jnp.concatenate(
        idx_rows + [jnp.zeros((3, bb), jnp.int32)], axis=0)


def kernel(inner_latent, outer_latent, face_grids, W, b, edge_index):
    B, D = inner_latent.shape
    T, F = face_grids.shape[0], face_grids.shape[1]

    E = edge_index.shape[1]
    EP = 512
    pad = jnp.full((EP - E,), -1, edge_index.dtype)
    src_f = jnp.broadcast_to(
        jnp.concatenate([edge_index[0], pad]).astype(jnp.float32)[None, :],
        (8, EP))
    dst_f = jnp.broadcast_to(
        jnp.concatenate([edge_index[1], pad]).astype(jnp.float32)[None, :],
        (8, EP))
    b2 = jnp.broadcast_to(b.reshape(D, 1), (D, 128))

    node_t = pl.pallas_call(
        _node_kernel,
        grid=(1,),
        in_specs=[
            pl.BlockSpec((T, F, D), lambda i: (0, 0, 0)),
            pl.BlockSpec((D, F * D), lambda i: (0, 0)),
            pl.BlockSpec((D, 128), lambda i: (0, 0)),
            pl.BlockSpec((8, EP), lambda i: (0, 0)),
            pl.BlockSpec((8, EP), lambda i: (0, 0)),
        ],
        out_specs=pl.BlockSpec((D, T), lambda i: (0, 0)),
        out_shape=jax.ShapeDtypeStruct((D, T), jnp.float32),
        scratch_shapes=[pltpu.VMEM((D, T), jnp.float32)],
        compiler_params=pltpu.CompilerParams(
            dimension_semantics=("arbitrary",)),
    )(face_grids, W, b2, src_f, dst_f)

    BB = 512
    nb = B // BB
    ones_row = jnp.ones((8, D), jnp.float32)
    vals8, idx8 = pl.pallas_call(
        _score_topk_kernel,
        grid=(nb,),
        in_specs=[
            pl.BlockSpec((BB, D), lambda i: (i, 0)),
            pl.BlockSpec((BB, D), lambda i: (i, 0)),
            pl.BlockSpec((D, T), lambda i: (0, 0)),
            pl.BlockSpec((8, D), lambda i: (0, 0)),
        ],
        out_specs=[
            pl.BlockSpec((8, BB), lambda i: (0, i)),
            pl.BlockSpec((8, BB), lambda i: (0, i)),
        ],
        out_shape=[
            jax.ShapeDtypeStruct((8, B), jnp.float32),
            jax.ShapeDtypeStruct((8, B), jnp.int32),
        ],
        compiler_params=pltpu.CompilerParams(
            dimension_semantics=("parallel",)),
    )(inner_latent, outer_latent, node_t, ones_row)

    return vals8[:5].T, idx8[:5].T
